# baseline (speedup 1.0000x reference)
import jax
import jax.numpy as jnp
from jax import lax
from jax.experimental import pallas as pl
from jax.experimental.pallas import tpu as pltpu
from jax.experimental.pallas import tpu_sc as plsc

N = 10000
DH = 128
HF = 64
E = 320000

NC = 2
NS = 16
NW = NC * NS

CHUNK = 128
NCH = 80
A = 10240
RPS = A // NS
TRASH = N

_mesh = plsc.VectorSubcoreMesh(
    core_axis_name="c", subcore_axis_name="s", num_cores=NC, num_subcores=NS)


def _lk(h):
    return jnp.where(h >= 0, h, 0.01 * h)


def _deg_body(dst_hbm, out_hbm, dst_v, cnt_v):
    c = lax.axis_index("c")
    s = lax.axis_index("s")
    w = s * NC + c
    pltpu.sync_copy(dst_hbm.at[w], dst_v)

    def zbody(i, carry):
        cnt_v[pl.ds(i * 16, 16)] = jnp.zeros((16,), jnp.float32)
        return carry

    lax.fori_loop(0, A // 16, zbody, 0)
    ones = jnp.ones((16,), jnp.float32)

    def body(j, carry):
        def inner(l, carry2):
            idx = dst_v[pl.ds(j * CHUNK + l * 16, 16)]
            plsc.addupdate_scatter(cnt_v, [idx], ones)
            return carry2
        return lax.fori_loop(0, CHUNK // 16, inner, carry)

    lax.fori_loop(0, NCH, body, 0)
    pltpu.sync_copy(cnt_v, out_hbm.at[c, s])


_deg_call = pl.kernel(
    _deg_body,
    out_type=jax.ShapeDtypeStruct((NC, NS, A), jnp.float32),
    mesh=_mesh,
    compiler_params=pltpu.CompilerParams(needs_layout_passes=False),
    scratch_types=[
        pltpu.VMEM((NCH * CHUNK,), jnp.int32),
        pltpu.VMEM((A,), jnp.float32),
    ],
)


def _scatter_body(g_hbm, src_hbm, dst_hbm, zeros_hbm, out_hbm,
                  src_v, dst_v, rows_v, shared, sem):
    c = lax.axis_index("c")
    s = lax.axis_index("s")
    w = s * NC + c
    pltpu.sync_copy(zeros_hbm, shared.at[pl.ds(s * RPS, RPS)])
    pltpu.sync_copy(src_hbm.at[w], src_v)
    pltpu.sync_copy(dst_hbm.at[w], dst_v)
    plsc.subcore_barrier()

    def body(j, carry):
        pltpu.async_copy(g_hbm.at[src_v.at[j]], rows_v, sem).wait()
        pltpu.sync_copy(rows_v, shared.at[dst_v.at[j]], add=True)
        return carry

    lax.fori_loop(0, NCH, body, 0)
    plsc.subcore_barrier()
    pltpu.sync_copy(shared.at[pl.ds(s * RPS, RPS)],
                    out_hbm.at[c, pl.ds(s * RPS, RPS)])


_scatter_call = pl.kernel(
    _scatter_body,
    out_type=jax.ShapeDtypeStruct((NC, A, DH), jnp.float32),
    mesh=_mesh,
    scratch_types=[
        pltpu.VMEM((NCH, CHUNK), jnp.int32),
        pltpu.VMEM((NCH, CHUNK), jnp.int32),
        pltpu.VMEM((CHUNK, DH), jnp.float32),
        pltpu.VMEM_SHARED((A, DH), jnp.float32),
        pltpu.SemaphoreType.DMA,
    ],
)


BR = 1024
NB = -(-N // BR)

_row = pl.BlockSpec((BR, DH), lambda i: (i, 0))
_mat = pl.BlockSpec((DH, DH), lambda i: (0, 0))
_vec = pl.BlockSpec((1, DH), lambda i: (0, 0))
_par = pl.BlockSpec((NC, BR, DH), lambda i: (0, i, 0))
_degblk = pl.BlockSpec((NC, NS, BR), lambda i: (0, 0, i))


def _mlp_body(x_ref, w1_ref, b1_ref, w2_ref, b2_ref, o_ref):
    h = _lk(jnp.dot(x_ref[...], w1_ref[...],
                    preferred_element_type=jnp.float32) + b1_ref[...])
    o_ref[...] = _lk(jnp.dot(h, w2_ref[...],
                             preferred_element_type=jnp.float32) + b2_ref[...])


_mlp_call = pl.pallas_call(
    _mlp_body, grid=(NB,),
    in_specs=[_row, _mat, _vec, _mat, _vec],
    out_specs=_row,
    out_shape=jax.ShapeDtypeStruct((N, DH), jnp.float32),
)


def _scale_body(degp_ref, h1_ref, wc_ref, g_ref, dinv_ref):
    deg = 1.0 + jnp.sum(degp_ref[...], axis=(0, 1))
    dinv = lax.rsqrt(deg)[:, None]
    hw = jnp.dot(h1_ref[...], wc_ref[...], preferred_element_type=jnp.float32)
    g_ref[...] = hw * dinv
    dinv_ref[...] = jnp.broadcast_to(dinv, (BR, DH))


_scale_call = pl.pallas_call(
    _scale_body, grid=(NB,),
    in_specs=[_degblk, _row, _mat],
    out_specs=[_row, _row],
    out_shape=[jax.ShapeDtypeStruct((N, DH), jnp.float32),
               jax.ShapeDtypeStruct((N, DH), jnp.float32)],
)


def _post_body(sp_ref, g_ref, dinv_ref, b_ref, w_ref, h_ref, gn_ref):
    ssum = sp_ref[0] + sp_ref[1]
    h = _lk(dinv_ref[...] * (ssum + g_ref[...]) + b_ref[...])
    h_ref[...] = h
    gn_ref[...] = jnp.dot(h, w_ref[...],
                          preferred_element_type=jnp.float32) * dinv_ref[...]


_post_call = pl.pallas_call(
    _post_body, grid=(NB,),
    in_specs=[_par, _row, _row, _vec, _mat],
    out_specs=[_row, _row],
    out_shape=[jax.ShapeDtypeStruct((N, DH), jnp.float32),
               jax.ShapeDtypeStruct((N, DH), jnp.float32)],
)


def _final_body(sp_ref, g_ref, dinv_ref, b_ref, wp_ref, bp_ref, h_ref, o_ref):
    ssum = sp_ref[0] + sp_ref[1]
    h = _lk(dinv_ref[...] * (ssum + g_ref[...]) + b_ref[...])
    h_ref[...] = h
    o_ref[...] = jnp.dot(h, wp_ref[...],
                         preferred_element_type=jnp.float32) + bp_ref[...]


_final_call = pl.pallas_call(
    _final_body, grid=(NB,),
    in_specs=[_par, _row, _row, _vec, _mat, _vec],
    out_specs=[_row, _row],
    out_shape=[jax.ShapeDtypeStruct((N, DH), jnp.float32),
               jax.ShapeDtypeStruct((N, DH), jnp.float32)],
)


@jax.jit
def kernel(x, edge_index, W1, b1, W2, b2, Wc1, bc1, Wc2, bc2, Wp, bp):
    src = edge_index[0]
    dst = edge_index[1]
    epw = E // NW
    padn = NCH * CHUNK - epw
    pad_src = jnp.zeros((NW, padn), jnp.int32)
    pad_dst = (TRASH
               + (jnp.arange(padn, dtype=jnp.int32)[None, :]
                  + 7 * jnp.arange(NW, dtype=jnp.int32)[:, None]) % (A - N))
    src_p = jnp.concatenate(
        [src.reshape(NW, epw), pad_src], axis=1).reshape(NW, NCH, CHUNK)
    dst_p = jnp.concatenate(
        [dst.reshape(NW, epw), pad_dst], axis=1).reshape(NW, NCH, CHUNK)

    zeros128 = jnp.zeros((RPS, DH), jnp.float32)

    b1r = b1.reshape(1, DH)
    b2r = b2.reshape(1, DH)
    bc1r = bc1.reshape(1, DH)
    bc2r = bc2.reshape(1, DH)
    wp_pad = jnp.zeros((DH, DH), jnp.float32).at[:, 0].set(Wp[:, 0])
    bp_pad = jnp.zeros((1, DH), jnp.float32) + bp[0]

    degp = _deg_call(dst_p.reshape(NW, NCH * CHUNK))
    h1 = _mlp_call(x, W1, b1r, W2, b2r)
    g1, dinv = _scale_call(degp, h1, Wc1)
    s1 = _scatter_call(g1, src_p, dst_p, zeros128)
    h2, g2 = _post_call(s1, g1, dinv, bc1r, Wc2)
    s2 = _scatter_call(g2, src_p, dst_p, zeros128)
    h3, ocol = _final_call(s2, g2, dinv, bc2r, wp_pad, bp_pad)
    return ocol[:, 0], h2, h3


# speedup vs baseline: 1.2898x; 1.2898x over previous
import jax
import jax.numpy as jnp
from jax import lax
from jax.experimental import pallas as pl
from jax.experimental.pallas import tpu as pltpu
from jax.experimental.pallas import tpu_sc as plsc

N = 10000
DH = 128
HF = 64
E = 320000

NC = 2
NS = 16
NW = NC * NS

CHUNK = 128
NCH = 79
EP = NW * NCH * CHUNK
A = 10240
RPS = A // NS
TRASH = N

_mesh = plsc.VectorSubcoreMesh(
    core_axis_name="c", subcore_axis_name="s", num_cores=NC, num_subcores=NS)


def _lk(h):
    return jnp.where(h >= 0, h, 0.01 * h)


def _deg_body(dst_hbm, out_hbm, dst_v, cnt_v):
    c = lax.axis_index("c")
    s = lax.axis_index("s")
    w = s * NC + c
    pltpu.sync_copy(dst_hbm.at[w], dst_v)

    def zbody(i, carry):
        cnt_v[pl.ds(i * 16, 16)] = jnp.zeros((16,), jnp.float32)
        return carry

    lax.fori_loop(0, A // 16, zbody, 0)
    ones = jnp.ones((16,), jnp.float32)

    def body(j, carry):
        def inner(l, carry2):
            idx = dst_v[pl.ds(j * CHUNK + l * 16, 16)]
            plsc.addupdate_scatter(cnt_v, [idx], ones)
            return carry2
        return lax.fori_loop(0, CHUNK // 16, inner, carry)

    lax.fori_loop(0, NCH, body, 0)
    pltpu.sync_copy(cnt_v, out_hbm.at[c, s])


_deg_call = pl.kernel(
    _deg_body,
    out_type=jax.ShapeDtypeStruct((NC, NS, A), jnp.float32),
    mesh=_mesh,
    compiler_params=pltpu.CompilerParams(needs_layout_passes=False),
    scratch_types=[
        pltpu.VMEM((NCH * CHUNK,), jnp.int32),
        pltpu.VMEM((A,), jnp.float32),
    ],
)


def _scatter_body(g_hbm, src_hbm, dst_hbm, zeros_hbm, out_hbm,
                  src_v, dst_v, rows_v, shared, sem):
    c = lax.axis_index("c")
    s = lax.axis_index("s")
    w = s * NC + c
    pltpu.sync_copy(zeros_hbm, shared.at[pl.ds(s * RPS, RPS)])
    pltpu.sync_copy(src_hbm.at[w], src_v)
    pltpu.sync_copy(dst_hbm.at[w], dst_v)
    plsc.subcore_barrier()

    def body(j, carry):
        pltpu.async_copy(g_hbm.at[src_v.at[j]], rows_v, sem).wait()
        pltpu.sync_copy(rows_v, shared.at[dst_v.at[j]], add=True)
        return carry

    lax.fori_loop(0, NCH, body, 0)
    plsc.subcore_barrier()
    pltpu.sync_copy(shared.at[pl.ds(s * RPS, RPS)],
                    out_hbm.at[c, pl.ds(s * RPS, RPS)])


_scatter_call = pl.kernel(
    _scatter_body,
    out_type=jax.ShapeDtypeStruct((NC, A, DH), jnp.float32),
    mesh=_mesh,
    scratch_types=[
        pltpu.VMEM((NCH, CHUNK), jnp.int32),
        pltpu.VMEM((NCH, CHUNK), jnp.int32),
        pltpu.VMEM((CHUNK, DH), jnp.float32),
        pltpu.VMEM_SHARED((A, DH), jnp.float32),
        pltpu.SemaphoreType.DMA,
    ],
)


BR = 1024
NB = -(-N // BR)

_row = pl.BlockSpec((BR, DH), lambda i: (i, 0))
_mat = pl.BlockSpec((DH, DH), lambda i: (0, 0))
_vec = pl.BlockSpec((1, DH), lambda i: (0, 0))
_par = pl.BlockSpec((NC, BR, DH), lambda i: (0, i, 0))
_degblk = pl.BlockSpec((NC, NS, BR), lambda i: (0, 0, i))


def _mlp_body(x_ref, w1_ref, b1_ref, w2_ref, b2_ref, o_ref):
    h = _lk(jnp.dot(x_ref[...], w1_ref[...],
                    preferred_element_type=jnp.float32) + b1_ref[...])
    o_ref[...] = _lk(jnp.dot(h, w2_ref[...],
                             preferred_element_type=jnp.float32) + b2_ref[...])


_mlp_call = pl.pallas_call(
    _mlp_body, grid=(NB,),
    in_specs=[_row, _mat, _vec, _mat, _vec],
    out_specs=_row,
    out_shape=jax.ShapeDtypeStruct((N, DH), jnp.float32),
)


def _scale_body(degp_ref, h1_ref, wc_ref, g_ref, dinv_ref):
    deg = 1.0 + jnp.sum(degp_ref[...], axis=(0, 1))
    dinv = lax.rsqrt(deg)[:, None]
    hw = jnp.dot(h1_ref[...], wc_ref[...], preferred_element_type=jnp.float32)
    g_ref[...] = hw * dinv
    dinv_ref[...] = jnp.broadcast_to(dinv, (BR, DH))


_scale_call = pl.pallas_call(
    _scale_body, grid=(NB,),
    in_specs=[_degblk, _row, _mat],
    out_specs=[_row, _row],
    out_shape=[jax.ShapeDtypeStruct((N, DH), jnp.float32),
               jax.ShapeDtypeStruct((N, DH), jnp.float32)],
)


def _post_body(sp_ref, g_ref, dinv_ref, b_ref, w_ref, h_ref, gn_ref):
    ssum = sp_ref[0] + sp_ref[1]
    h = _lk(dinv_ref[...] * (ssum + g_ref[...]) + b_ref[...])
    h_ref[...] = h
    gn_ref[...] = jnp.dot(h, w_ref[...],
                          preferred_element_type=jnp.float32) * dinv_ref[...]


_post_call = pl.pallas_call(
    _post_body, grid=(NB,),
    in_specs=[_par, _row, _row, _vec, _mat],
    out_specs=[_row, _row],
    out_shape=[jax.ShapeDtypeStruct((N, DH), jnp.float32),
               jax.ShapeDtypeStruct((N, DH), jnp.float32)],
)


def _final_body(sp_ref, g_ref, dinv_ref, b_ref, wp_ref, bp_ref, h_ref, o_ref):
    ssum = sp_ref[0] + sp_ref[1]
    h = _lk(dinv_ref[...] * (ssum + g_ref[...]) + b_ref[...])
    h_ref[...] = h
    o_ref[...] = jnp.dot(h, wp_ref[...],
                         preferred_element_type=jnp.float32) + bp_ref[...]


_final_call = pl.pallas_call(
    _final_body, grid=(NB,),
    in_specs=[_par, _row, _row, _vec, _mat, _vec],
    out_specs=[_row, _row],
    out_shape=[jax.ShapeDtypeStruct((N, DH), jnp.float32),
               jax.ShapeDtypeStruct((N, DH), jnp.float32)],
)


@jax.jit
def kernel(x, edge_index, W1, b1, W2, b2, Wc1, bc1, Wc2, bc2, Wp, bp):
    src = edge_index[0]
    dst = edge_index[1]
    padn = EP - E
    pad_src = jnp.zeros((padn,), jnp.int32)
    pad_dst = TRASH + (jnp.arange(padn, dtype=jnp.int32) % (A - N))
    src_p = jnp.concatenate([src, pad_src]).reshape(NW, NCH, CHUNK)
    dst_p = jnp.concatenate([dst, pad_dst]).reshape(NW, NCH, CHUNK)

    zeros128 = jnp.zeros((RPS, DH), jnp.float32)

    b1r = b1.reshape(1, DH)
    b2r = b2.reshape(1, DH)
    bc1r = bc1.reshape(1, DH)
    bc2r = bc2.reshape(1, DH)
    wp_pad = jnp.zeros((DH, DH), jnp.float32).at[:, 0].set(Wp[:, 0])
    bp_pad = jnp.zeros((1, DH), jnp.float32) + bp[0]

    degp = _deg_call(dst_p.reshape(NW, NCH * CHUNK))
    h1 = _mlp_call(x, W1, b1r, W2, b2r)
    g1, dinv = _scale_call(degp, h1, Wc1)
    s1 = _scatter_call(g1, src_p, dst_p, zeros128)
    h2, g2 = _post_call(s1, g1, dinv, bc1r, Wc2)
    s2 = _scatter_call(g2, src_p, dst_p, zeros128)
    h3, ocol = _final_call(s2, g2, dinv, bc2r, wp_pad, bp_pad)
    return ocol[:, 0], h2, h3


# speedup vs baseline: 1.2910x; 1.0009x over previous
import jax
import jax.numpy as jnp
from jax import lax
from jax.experimental import pallas as pl
from jax.experimental.pallas import tpu as pltpu
from jax.experimental.pallas import tpu_sc as plsc

N = 10000
DH = 128
E = 320000

NC = 2
NS = 16
NW = NC * NS

CHUNK = 128
NCH = 79
EP = NW * NCH * CHUNK
A = 10240
RPS = A // NS
TRASH = N

_mesh = plsc.VectorSubcoreMesh(
    core_axis_name="c", subcore_axis_name="s", num_cores=NC, num_subcores=NS)


def _lk(h):
    return jnp.where(h >= 0, h, 0.01 * h)


def _deg_body(dst_hbm, out_hbm, dst_v, cnt_v):
    c = lax.axis_index("c")
    s = lax.axis_index("s")
    w = s * NC + c
    pltpu.sync_copy(dst_hbm.at[w], dst_v)

    def zbody(i, carry):
        cnt_v[pl.ds(i * 16, 16)] = jnp.zeros((16,), jnp.float32)
        return carry

    lax.fori_loop(0, A // 16, zbody, 0)
    ones = jnp.ones((16,), jnp.float32)

    def body(j, carry):
        def inner(l, carry2):
            idx = dst_v[pl.ds(j * CHUNK + l * 16, 16)]
            plsc.addupdate_scatter(cnt_v, [idx], ones)
            return carry2
        return lax.fori_loop(0, CHUNK // 16, inner, carry)

    lax.fori_loop(0, NCH, body, 0)
    pltpu.sync_copy(cnt_v, out_hbm.at[c, s])


_deg_call = pl.kernel(
    _deg_body,
    out_type=jax.ShapeDtypeStruct((NC, NS, A), jnp.float32),
    mesh=_mesh,
    compiler_params=pltpu.CompilerParams(needs_layout_passes=False),
    scratch_types=[
        pltpu.VMEM((NCH * CHUNK,), jnp.int32),
        pltpu.VMEM((A,), jnp.float32),
    ],
)


def _scatter_body(g_hbm, src_hbm, dst_hbm, zeros_hbm, out_hbm,
                  src_v, dst_v, rows_v, shared, sem):
    c = lax.axis_index("c")
    s = lax.axis_index("s")
    w = s * NC + c
    pltpu.sync_copy(zeros_hbm, shared.at[pl.ds(s * RPS, RPS)])
    pltpu.sync_copy(src_hbm.at[w], src_v)
    pltpu.sync_copy(dst_hbm.at[w], dst_v)
    plsc.subcore_barrier()

    def body(j, carry):
        pltpu.async_copy(g_hbm.at[src_v.at[j]], rows_v, sem).wait()
        pltpu.sync_copy(rows_v, shared.at[dst_v.at[j]], add=True)
        return carry

    lax.fori_loop(0, NCH, body, 0)
    plsc.subcore_barrier()
    pltpu.sync_copy(shared.at[pl.ds(s * RPS, RPS)],
                    out_hbm.at[c, pl.ds(s * RPS, RPS)])


_scatter_call = pl.kernel(
    _scatter_body,
    out_type=jax.ShapeDtypeStruct((NC, A, DH), jnp.float32),
    mesh=_mesh,
    scratch_types=[
        pltpu.VMEM((NCH, CHUNK), jnp.int32),
        pltpu.VMEM((NCH, CHUNK), jnp.int32),
        pltpu.VMEM((CHUNK, DH), jnp.float32),
        pltpu.VMEM_SHARED((A, DH), jnp.float32),
        pltpu.SemaphoreType.DMA,
    ],
)


BR = 1024
NB = -(-N // BR)

_row = pl.BlockSpec((BR, DH), lambda i: (i, 0))
_mat = pl.BlockSpec((DH, DH), lambda i: (0, 0))
_vec = pl.BlockSpec((1, DH), lambda i: (0, 0))
_par = pl.BlockSpec((NC, BR, DH), lambda i: (0, i, 0))
_degblk = pl.BlockSpec((NC, NS, BR), lambda i: (0, 0, i))


def _mlp_body(x_ref, w1_ref, b1_ref, w2_ref, b2_ref, o_ref):
    h = _lk(jnp.dot(x_ref[...], w1_ref[...],
                    preferred_element_type=jnp.float32) + b1_ref[...])
    o_ref[...] = _lk(jnp.dot(h, w2_ref[...],
                             preferred_element_type=jnp.float32) + b2_ref[...])


_mlp_call = pl.pallas_call(
    _mlp_body, grid=(NB,),
    in_specs=[_row, _mat, _vec, _mat, _vec],
    out_specs=_row,
    out_shape=jax.ShapeDtypeStruct((N, DH), jnp.float32),
)


def _scale_body(degp_ref, h1_ref, wc_ref, g_ref, dinv_ref):
    deg = 1.0 + jnp.sum(degp_ref[...], axis=(0, 1))
    dinv = lax.rsqrt(deg)[:, None]
    hw = jnp.dot(h1_ref[...], wc_ref[...], preferred_element_type=jnp.float32)
    g_ref[...] = hw * dinv
    dinv_ref[...] = jnp.broadcast_to(dinv, (BR, DH))


_scale_call = pl.pallas_call(
    _scale_body, grid=(NB,),
    in_specs=[_degblk, _row, _mat],
    out_specs=[_row, _row],
    out_shape=[jax.ShapeDtypeStruct((N, DH), jnp.float32),
               jax.ShapeDtypeStruct((N, DH), jnp.float32)],
)


def _post_body(sp_ref, g_ref, dinv_ref, b_ref, w_ref, h_ref, gn_ref):
    ssum = sp_ref[0] + sp_ref[1]
    h = _lk(dinv_ref[...] * (ssum + g_ref[...]) + b_ref[...])
    h_ref[...] = h
    gn_ref[...] = jnp.dot(h, w_ref[...],
                          preferred_element_type=jnp.float32) * dinv_ref[...]


_post_call = pl.pallas_call(
    _post_body, grid=(NB,),
    in_specs=[_par, _row, _row, _vec, _mat],
    out_specs=[_row, _row],
    out_shape=[jax.ShapeDtypeStruct((N, DH), jnp.float32),
               jax.ShapeDtypeStruct((N, DH), jnp.float32)],
)


def _final_body(sp_ref, g_ref, dinv_ref, b_ref, wp_ref, bp_ref, h_ref, o_ref):
    ssum = sp_ref[0] + sp_ref[1]
    h = _lk(dinv_ref[...] * (ssum + g_ref[...]) + b_ref[...])
    h_ref[...] = h
    o_ref[...] = jnp.dot(h, wp_ref[...],
                         preferred_element_type=jnp.float32) + bp_ref[...]


_final_call = pl.pallas_call(
    _final_body, grid=(NB,),
    in_specs=[_par, _row, _row, _vec, _mat, _vec],
    out_specs=[_row, _row],
    out_shape=[jax.ShapeDtypeStruct((N, DH), jnp.float32),
               jax.ShapeDtypeStruct((N, DH), jnp.float32)],
)


@jax.jit
def kernel(x, edge_index, W1, b1, W2, b2, Wc1, bc1, Wc2, bc2, Wp, bp):
    src = edge_index[0]
    dst = edge_index[1]
    padn = EP - E
    pad_src = jnp.zeros((padn,), jnp.int32)
    pad_dst = TRASH + (jnp.arange(padn, dtype=jnp.int32) % (A - N))
    src_p = jnp.concatenate([src, pad_src]).reshape(NW, NCH, CHUNK)
    dst_p = jnp.concatenate([dst, pad_dst]).reshape(NW, NCH, CHUNK)

    zeros128 = jnp.zeros((RPS, DH), jnp.float32)

    b1r = b1.reshape(1, DH)
    b2r = b2.reshape(1, DH)
    bc1r = bc1.reshape(1, DH)
    bc2r = bc2.reshape(1, DH)
    wp_pad = jnp.zeros((DH, DH), jnp.float32).at[:, 0].set(Wp[:, 0])
    bp_pad = jnp.zeros((1, DH), jnp.float32) + bp[0]

    degp = _deg_call(dst_p.reshape(NW, NCH * CHUNK))
    h1 = _mlp_call(x, W1, b1r, W2, b2r)
    g1, dinv = _scale_call(degp, h1, Wc1)
    s1 = _scatter_call(g1, src_p, dst_p, zeros128)
    h2, g2 = _post_call(s1, g1, dinv, bc1r, Wc2)
    s2 = _scatter_call(g2, src_p, dst_p, zeros128)
    h3, ocol = _final_call(s2, g2, dinv, bc2r, wp_pad, bp_pad)
    return ocol[:, 0], h2, h3
